# Initial kernel scaffold; baseline (speedup 1.0000x reference)
import jax
import jax.numpy as jnp
from jax.experimental import pallas as pl


def kernel(rf, t0, d_tx, d_rx, fs, c0, apod):
    raise NotImplementedError("write your pallas kernel here")


# speedup vs baseline: 774.6699x; 774.6699x over previous
import functools

import jax
import jax.numpy as jnp
from jax import lax
from jax.experimental import pallas as pl
from jax.experimental.pallas import tpu as pltpu
from jax.experimental.pallas import tpu_sc as plsc

NC = 2
NS = 16
L = 16
NW = NC * NS

A = 8
E = 128
NSAMP = 2048
P = 512 * 256
CHUNK = P // NW
TBL = A * NSAMP


def _das_sc(rf2, dtx_s, drx2, apod2, kvec):
    mesh = plsc.VectorSubcoreMesh(
        core_axis_name="c", subcore_axis_name="s", num_cores=NC, num_subcores=NS
    )

    @functools.partial(
        pl.kernel,
        out_type=jax.ShapeDtypeStruct((A, P), jnp.float32),
        mesh=mesh,
        compiler_params=pltpu.CompilerParams(needs_layout_passes=False),
        scratch_types=[
            pltpu.VMEM((TBL + L,), jnp.float32),
            pltpu.VMEM((A, CHUNK), jnp.float32),
            pltpu.VMEM((CHUNK,), jnp.float32),
            pltpu.VMEM((CHUNK,), jnp.float32),
            pltpu.VMEM((A, CHUNK), jnp.float32),
            pltpu.VMEM((L,), jnp.float32),
        ],
    )
    def k(rf_hbm, dtx_hbm, drx_hbm, apod_hbm, kvec_hbm, out_hbm,
          rf_v, dtx_v, drx_v, apod_v, acc_v, k_v):
        wid = lax.axis_index("s") * NC + lax.axis_index("c")
        base = wid * CHUNK

        pltpu.sync_copy(kvec_hbm, k_v)
        pltpu.sync_copy(dtx_hbm.at[:, pl.ds(base, CHUNK)], dtx_v)
        kreg = k_v[...]

        zeros = jnp.zeros((L,), jnp.float32)

        def zero_body(j, _):
            px = pl.ds(j * L, L)
            for a in range(A):
                acc_v[a, px] = zeros
            return 0

        lax.fori_loop(0, CHUNK // L, zero_body, 0)
        rf_v[pl.ds(TBL, L)] = zeros

        def elem_body(e, _):
            pltpu.sync_copy(rf_hbm.at[e], rf_v.at[pl.ds(0, TBL)])
            pltpu.sync_copy(drx_hbm.at[e, pl.ds(base, CHUNK)], drx_v)
            pltpu.sync_copy(apod_hbm.at[e, pl.ds(base, CHUNK)], apod_v)

            def px_body(i, _):
                px = pl.ds(i * L, L)
                drxk = drx_v[px] * kreg
                ap = apod_v[px]
                for a in range(A):
                    d = dtx_v[a, px] + drxk
                    i0 = d.astype(jnp.int32)
                    w = d - i0.astype(jnp.float32)
                    v0 = plsc.load_gather(rf_v, [i0])
                    v1 = plsc.load_gather(rf_v, [i0 + 1])
                    acc_v[a, px] = acc_v[a, px] + (v0 + w * (v1 - v0)) * ap
                return 0

            lax.fori_loop(0, CHUNK // L, px_body, 0)
            return 0

        lax.fori_loop(0, E, elem_body, 0)
        pltpu.sync_copy(acc_v, out_hbm.at[:, pl.ds(base, CHUNK)])

    return k(rf2, dtx_s, drx2, apod2, kvec)


def kernel(rf, t0, d_tx, d_rx, fs, c0, apod):
    n_angles, n_elements, n_samp = rf.shape
    _, nz, nx = d_tx.shape
    kscale = (fs / c0).astype(jnp.float32)
    offs = (jnp.arange(n_angles, dtype=jnp.float32) * n_samp
            - t0.astype(jnp.float32) * fs)
    dtx_s = d_tx.reshape(n_angles, nz * nx) * kscale + offs[:, None]
    rf2 = rf.transpose(1, 0, 2).reshape(n_elements, n_angles * n_samp)
    drx2 = d_rx.reshape(n_elements, nz * nx)
    apod2 = apod.reshape(n_elements, nz * nx)
    kvec = jnp.full((L,), kscale, jnp.float32)
    out = _das_sc(rf2, dtx_s, drx2, apod2, kvec)
    return out.reshape(n_angles, nz, nx)


# speedup vs baseline: 3477.2582x; 4.4887x over previous
import functools

import jax
import jax.numpy as jnp
from jax import lax
from jax.experimental import pallas as pl
from jax.experimental.pallas import tpu as pltpu
from jax.experimental.pallas import tpu_sc as plsc

NC = 2
NS = 16
L = 16
NW = NC * NS

A = 8
E = 128
NSAMP = 2048
P = 512 * 256
CHUNK = P // NW
HALF = CHUNK // 2
TBL = A * NSAMP
EB = 2
NBLK = E // EB
BTBL = EB * TBL


def _das_sc(rf_flat, dtx_s, drx2, apod2, kvec):
    mesh = plsc.VectorSubcoreMesh(
        core_axis_name="c", subcore_axis_name="s", num_cores=NC, num_subcores=NS
    )

    @functools.partial(
        pl.kernel,
        out_type=jax.ShapeDtypeStruct((A, P), jnp.float32),
        mesh=mesh,
        compiler_params=pltpu.CompilerParams(needs_layout_passes=False),
        scratch_types=[
            pltpu.VMEM((BTBL + L,), jnp.float32),
            pltpu.VMEM((BTBL + L,), jnp.float32),
            pltpu.VMEM((2, EB, HALF), jnp.float32),
            pltpu.VMEM((2, EB, HALF), jnp.float32),
            pltpu.VMEM((A, HALF), jnp.float32),
            pltpu.VMEM((A, HALF), jnp.float32),
            pltpu.VMEM((L,), jnp.float32),
            pltpu.SemaphoreType.DMA,
            pltpu.SemaphoreType.DMA,
            pltpu.SemaphoreType.DMA,
            pltpu.SemaphoreType.DMA,
            pltpu.SemaphoreType.DMA,
            pltpu.SemaphoreType.DMA,
        ],
    )
    def k(rf_hbm, dtx_hbm, drx_hbm, apod_hbm, kvec_hbm, out_hbm,
          rf_v0, rf_v1, drx_v, apod_v, dtx_v, acc_v, k_v,
          rf_s0, rf_s1, drx_s0, drx_s1, ap_s0, ap_s1):
        wid = lax.axis_index("s") * NC + lax.axis_index("c")
        rf_bufs = (rf_v0, rf_v1)
        rf_sems = (rf_s0, rf_s1)
        drx_sems = (drx_s0, drx_s1)
        ap_sems = (ap_s0, ap_s1)

        pltpu.sync_copy(kvec_hbm, k_v)
        kreg = k_v[...]
        zeros = jnp.zeros((L,), jnp.float32)
        for b in (0, 1):
            rf_bufs[b][pl.ds(BTBL, L)] = zeros

        def start_block(eb, b):
            pltpu.async_copy(
                rf_hbm.at[pl.ds(eb * BTBL, BTBL)],
                rf_bufs[b].at[pl.ds(0, BTBL)], rf_sems[b])
            pltpu.async_copy(
                drx_hbm.at[pl.ds(eb * EB, EB), pl.ds(pbase, HALF)],
                drx_v.at[b], drx_sems[b])
            pltpu.async_copy(
                apod_hbm.at[pl.ds(eb * EB, EB), pl.ds(pbase, HALF)],
                apod_v.at[b], ap_sems[b])

        def wait_block(eb, b):
            pltpu.make_async_copy(
                rf_hbm.at[pl.ds(eb * BTBL, BTBL)],
                rf_bufs[b].at[pl.ds(0, BTBL)], rf_sems[b]).wait()
            pltpu.make_async_copy(
                drx_hbm.at[pl.ds(eb * EB, EB), pl.ds(pbase, HALF)],
                drx_v.at[b], drx_sems[b]).wait()
            pltpu.make_async_copy(
                apod_hbm.at[pl.ds(eb * EB, EB), pl.ds(pbase, HALF)],
                apod_v.at[b], ap_sems[b]).wait()

        for half in range(2):
            pbase = wid * CHUNK + half * HALF
            pltpu.sync_copy(dtx_hbm.at[:, pl.ds(pbase, HALF)], dtx_v)

            @plsc.parallel_loop(0, HALF // L)
            def _zero(i):
                px = pl.ds(i * L, L)
                for a in range(A):
                    acc_v[a, px] = zeros

            start_block(0, 0)
            start_block(1, 1)

            def bb_body(bb, _):
                for b in (0, 1):
                    eb = bb * 2 + b
                    wait_block(eb, b)

                    @plsc.parallel_loop(0, HALF // L)
                    def _px(i):
                        px = pl.ds(i * L, L)
                        rf_b = rf_bufs[b]
                        dtx_regs = [dtx_v[a, px] for a in range(A)]
                        acc_regs = [acc_v[a, px] for a in range(A)]
                        for j in range(EB):
                            if j == 0:
                                drxk = drx_v[b, j, px] * kreg
                            else:
                                drxk = (drx_v[b, j, px] * kreg
                                        + jnp.float32(j * TBL))
                            ap = apod_v[b, j, px]
                            for a in range(A):
                                d = dtx_regs[a] + drxk
                                i0 = d.astype(jnp.int32)
                                w = d - i0.astype(jnp.float32)
                                v0 = plsc.load_gather(rf_b, [i0])
                                v1 = plsc.load_gather(rf_b, [i0 + 1])
                                acc_regs[a] = (acc_regs[a]
                                               + (v0 + w * (v1 - v0)) * ap)
                        for a in range(A):
                            acc_v[a, px] = acc_regs[a]

                    @pl.when(eb + 2 < NBLK)
                    def _():
                        start_block(eb + 2, b)
                return 0

            lax.fori_loop(0, NBLK // 2, bb_body, 0)
            pltpu.sync_copy(acc_v, out_hbm.at[:, pl.ds(pbase, HALF)])

    return k(rf_flat, dtx_s, drx2, apod2, kvec)


def kernel(rf, t0, d_tx, d_rx, fs, c0, apod):
    n_angles, n_elements, n_samp = rf.shape
    _, nz, nx = d_tx.shape
    kscale = (fs / c0).astype(jnp.float32)
    offs = (jnp.arange(n_angles, dtype=jnp.float32) * n_samp
            - t0.astype(jnp.float32) * fs)
    dtx_s = d_tx.reshape(n_angles, nz * nx) * kscale + offs[:, None]
    rf_flat = rf.transpose(1, 0, 2).reshape(n_elements * n_angles * n_samp)
    drx2 = d_rx.reshape(n_elements, nz * nx)
    apod2 = apod.reshape(n_elements, nz * nx)
    kvec = jnp.full((L,), kscale, jnp.float32)
    out = _das_sc(rf_flat, dtx_s, drx2, apod2, kvec)
    return out.reshape(n_angles, nz, nx)


# speedup vs baseline: 3728.3892x; 1.0722x over previous
import functools

import jax
import jax.numpy as jnp
from jax import lax
from jax.experimental import pallas as pl
from jax.experimental.pallas import tpu as pltpu
from jax.experimental.pallas import tpu_sc as plsc

NC = 2
NS = 16
L = 16
NW = NC * NS

A = 8
E = 128
NSAMP = 2048
P = 512 * 256
CHUNK = P // NW
HALF = CHUNK // 2
EB = 2
NBLK = E // EB
ATBL = EB * NSAMP
BTBL = A * ATBL


def _das_sc(rf_flat, dtx2, drx2, apod2, kvec, acst):
    mesh = plsc.VectorSubcoreMesh(
        core_axis_name="c", subcore_axis_name="s", num_cores=NC, num_subcores=NS
    )

    @functools.partial(
        pl.kernel,
        out_type=jax.ShapeDtypeStruct((A, P), jnp.float32),
        mesh=mesh,
        compiler_params=pltpu.CompilerParams(needs_layout_passes=False),
        scratch_types=[
            pltpu.VMEM((BTBL + L,), jnp.float32),
            pltpu.VMEM((BTBL + L,), jnp.float32),
            pltpu.VMEM((2, EB, HALF), jnp.float32),
            pltpu.VMEM((2, EB, HALF), jnp.float32),
            pltpu.VMEM((A, HALF), jnp.float32),
            pltpu.VMEM((A, HALF), jnp.float32),
            pltpu.VMEM((L,), jnp.float32),
            pltpu.VMEM((A, L), jnp.float32),
            pltpu.SemaphoreType.DMA,
            pltpu.SemaphoreType.DMA,
            pltpu.SemaphoreType.DMA,
            pltpu.SemaphoreType.DMA,
            pltpu.SemaphoreType.DMA,
            pltpu.SemaphoreType.DMA,
        ],
    )
    def k(rf_hbm, dtx_hbm, drx_hbm, apod_hbm, kvec_hbm, acst_hbm, out_hbm,
          rf_v0, rf_v1, drx_v, apod_v, dtx_v, acc_v, k_v, ac_v,
          rf_s0, rf_s1, drx_s0, drx_s1, ap_s0, ap_s1):
        wid = lax.axis_index("s") * NC + lax.axis_index("c")
        rf_bufs = (rf_v0, rf_v1)
        rf_sems = (rf_s0, rf_s1)
        drx_sems = (drx_s0, drx_s1)
        ap_sems = (ap_s0, ap_s1)

        pltpu.sync_copy(kvec_hbm, k_v)
        pltpu.sync_copy(acst_hbm, ac_v)
        kreg = k_v[...]
        zeros = jnp.zeros((L,), jnp.float32)
        for b in (0, 1):
            rf_bufs[b][pl.ds(BTBL, L)] = zeros

        def start_block(eb, b):
            for a in range(A):
                pltpu.async_copy(
                    rf_hbm.at[pl.ds(a * (E * NSAMP) + eb * ATBL, ATBL)],
                    rf_bufs[b].at[pl.ds(a * ATBL, ATBL)], rf_sems[b])
            pltpu.async_copy(
                drx_hbm.at[pl.ds(eb * EB, EB), pl.ds(pbase, HALF)],
                drx_v.at[b], drx_sems[b])
            pltpu.async_copy(
                apod_hbm.at[pl.ds(eb * EB, EB), pl.ds(pbase, HALF)],
                apod_v.at[b], ap_sems[b])

        def wait_block(eb, b):
            pltpu.make_async_copy(
                rf_hbm.at[pl.ds(0, BTBL)],
                rf_bufs[b].at[pl.ds(0, BTBL)], rf_sems[b]).wait()
            pltpu.make_async_copy(
                drx_hbm.at[pl.ds(eb * EB, EB), pl.ds(pbase, HALF)],
                drx_v.at[b], drx_sems[b]).wait()
            pltpu.make_async_copy(
                apod_hbm.at[pl.ds(eb * EB, EB), pl.ds(pbase, HALF)],
                apod_v.at[b], ap_sems[b]).wait()

        for half in range(2):
            pbase = wid * CHUNK + half * HALF
            pltpu.sync_copy(dtx_hbm.at[:, pl.ds(pbase, HALF)], dtx_v)

            @plsc.parallel_loop(0, HALF // L)
            def _pre(i):
                px = pl.ds(i * L, L)
                for a in range(A):
                    dtx_v[a, px] = dtx_v[a, px] * kreg + ac_v[a, :]
                    acc_v[a, px] = zeros

            start_block(0, 0)
            start_block(1, 1)

            def bb_body(bb, _):
                for b in (0, 1):
                    eb = bb * 2 + b
                    wait_block(eb, b)

                    @plsc.parallel_loop(0, HALF // L)
                    def _px(i):
                        px = pl.ds(i * L, L)
                        rf_b = rf_bufs[b]
                        dtx_regs = [dtx_v[a, px] for a in range(A)]
                        for j in range(EB):
                            if j == 0:
                                drxk = drx_v[b, j, px] * kreg
                            else:
                                drxk = (drx_v[b, j, px] * kreg
                                        + jnp.float32(j * NSAMP))
                            ap = apod_v[b, j, px]
                            for a in range(A):
                                d = dtx_regs[a] + drxk
                                i0 = d.astype(jnp.int32)
                                w = d - i0.astype(jnp.float32)
                                v0 = plsc.load_gather(rf_b, [i0])
                                v1 = plsc.load_gather(rf_b, [i0 + 1])
                                plsc.addupdate(acc_v.at[a, px],
                                               (v0 + w * (v1 - v0)) * ap)
                        return None

                    @pl.when(eb + 2 < NBLK)
                    def _():
                        start_block(eb + 2, b)
                return 0

            lax.fori_loop(0, NBLK // 2, bb_body, 0)
            pltpu.sync_copy(acc_v, out_hbm.at[:, pl.ds(pbase, HALF)])

    return k(rf_flat, dtx2, drx2, apod2, kvec, acst)


def kernel(rf, t0, d_tx, d_rx, fs, c0, apod):
    n_angles, n_elements, n_samp = rf.shape
    _, nz, nx = d_tx.shape
    kscale = (fs / c0).astype(jnp.float32)
    offs = (jnp.arange(n_angles, dtype=jnp.float32) * (EB * n_samp)
            - t0.astype(jnp.float32) * fs)
    acst = jnp.broadcast_to(offs[:, None], (n_angles, L))
    rf_flat = rf.reshape(n_angles * n_elements * n_samp)
    dtx2 = d_tx.reshape(n_angles, nz * nx)
    drx2 = d_rx.reshape(n_elements, nz * nx)
    apod2 = apod.reshape(n_elements, nz * nx)
    kvec = jnp.full((L,), kscale, jnp.float32)
    out = _das_sc(rf_flat, dtx2, drx2, apod2, kvec, acst)
    return out.reshape(n_angles, nz, nx)


# speedup vs baseline: 4392.3041x; 1.1781x over previous
import functools

import jax
import jax.numpy as jnp
from jax import lax
from jax.experimental import pallas as pl
from jax.experimental.pallas import tpu as pltpu
from jax.experimental.pallas import tpu_sc as plsc

NC = 2
NS = 16
L = 16
NW = NC * NS

A = 8
E = 128
NSAMP = 2048
NZ = 512
NX = 256
ZW = NZ // NW
ZH = ZW // 2
HALF = ZH * NX
EB = 2
NBLK = E // EB
ATBL = EB * NSAMP
BTBL = A * ATBL


def _das_sc(rf3, dtx3, drx3, apod3, kvec, acst):
    mesh = plsc.VectorSubcoreMesh(
        core_axis_name="c", subcore_axis_name="s", num_cores=NC, num_subcores=NS
    )

    @functools.partial(
        pl.kernel,
        out_type=jax.ShapeDtypeStruct((A, NZ, NX), jnp.float32),
        mesh=mesh,
        compiler_params=pltpu.CompilerParams(needs_layout_passes=False),
        scratch_types=[
            pltpu.VMEM((BTBL + L,), jnp.float32),
            pltpu.VMEM((BTBL + L,), jnp.float32),
            pltpu.VMEM((2, EB, ZH, NX), jnp.float32),
            pltpu.VMEM((2, EB, ZH, NX), jnp.float32),
            pltpu.VMEM((A, ZH, NX), jnp.float32),
            pltpu.VMEM((A, ZH, NX), jnp.float32),
            pltpu.VMEM((L,), jnp.float32),
            pltpu.VMEM((A, L), jnp.float32),
            pltpu.SemaphoreType.DMA,
            pltpu.SemaphoreType.DMA,
            pltpu.SemaphoreType.DMA,
            pltpu.SemaphoreType.DMA,
            pltpu.SemaphoreType.DMA,
            pltpu.SemaphoreType.DMA,
        ],
    )
    def k(rf_hbm, dtx_hbm, drx_hbm, apod_hbm, kvec_hbm, acst_hbm, out_hbm,
          rf_v0, rf_v1, drx_v, apod_v, dtx_v, acc_v, k_v, ac_v,
          rf_s0, rf_s1, drx_s0, drx_s1, ap_s0, ap_s1):
        wid = lax.axis_index("s") * NC + lax.axis_index("c")
        rf_bufs = (rf_v0, rf_v1)
        rf_sems = (rf_s0, rf_s1)
        drx_sems = (drx_s0, drx_s1)
        ap_sems = (ap_s0, ap_s1)

        pltpu.sync_copy(kvec_hbm, k_v)
        pltpu.sync_copy(acst_hbm, ac_v)
        kreg = k_v[...]
        zeros = jnp.zeros((L,), jnp.float32)
        for b in (0, 1):
            rf_bufs[b][pl.ds(BTBL, L)] = zeros

        def start_block(eb, b):
            for a in range(A):
                for j in range(EB):
                    pltpu.async_copy(
                        rf_hbm.at[a, eb * EB + j, :],
                        rf_bufs[b].at[pl.ds(a * ATBL + j * NSAMP, NSAMP)],
                        rf_sems[b])
            pltpu.async_copy(
                drx_hbm.at[pl.ds(eb * EB, EB), pl.ds(zb, ZH), :],
                drx_v.at[b], drx_sems[b])
            pltpu.async_copy(
                apod_hbm.at[pl.ds(eb * EB, EB), pl.ds(zb, ZH), :],
                apod_v.at[b], ap_sems[b])

        def wait_block(eb, b):
            for a in range(A):
                for j in range(EB):
                    pltpu.make_async_copy(
                        rf_hbm.at[a, eb * EB + j, :],
                        rf_bufs[b].at[pl.ds(a * ATBL + j * NSAMP, NSAMP)],
                        rf_sems[b]).wait()
            pltpu.make_async_copy(
                drx_hbm.at[pl.ds(eb * EB, EB), pl.ds(zb, ZH), :],
                drx_v.at[b], drx_sems[b]).wait()
            pltpu.make_async_copy(
                apod_hbm.at[pl.ds(eb * EB, EB), pl.ds(zb, ZH), :],
                apod_v.at[b], ap_sems[b]).wait()

        for half in range(2):
            zb = wid * ZW + half * ZH
            pltpu.sync_copy(dtx_hbm.at[:, pl.ds(zb, ZH), :], dtx_v)

            @plsc.parallel_loop(0, HALF // L)
            def _pre(i):
                zr = lax.shift_right_logical(i, 4)
                co = lax.shift_left(jnp.bitwise_and(i, 15), 4)
                px = pl.ds(co, L)
                for a in range(A):
                    dtx_v[a, zr, px] = dtx_v[a, zr, px] * kreg + ac_v[a, :]
                    acc_v[a, zr, px] = zeros

            start_block(0, 0)
            start_block(1, 1)

            def bb_body(bb, _):
                for b in (0, 1):
                    eb = bb * 2 + b
                    wait_block(eb, b)

                    @plsc.parallel_loop(0, HALF // L)
                    def _px(i):
                        zr = lax.shift_right_logical(i, 4)
                        co = lax.shift_left(jnp.bitwise_and(i, 15), 4)
                        px = pl.ds(co, L)
                        rf_b = rf_bufs[b]
                        dtx_regs = [dtx_v[a, zr, px] for a in range(A)]
                        for j in range(EB):
                            if j == 0:
                                drxk = drx_v[b, j, zr, px] * kreg
                            else:
                                drxk = (drx_v[b, j, zr, px] * kreg
                                        + jnp.float32(j * NSAMP))
                            ap = apod_v[b, j, zr, px]
                            for a in range(A):
                                d = dtx_regs[a] + drxk
                                i0 = d.astype(jnp.int32)
                                w = d - i0.astype(jnp.float32)
                                v0 = plsc.load_gather(rf_b, [i0])
                                v1 = plsc.load_gather(rf_b, [i0 + 1])
                                plsc.addupdate(acc_v.at[a, zr, px],
                                               (v0 + w * (v1 - v0)) * ap)
                        return None

                    @pl.when(eb + 2 < NBLK)
                    def _():
                        start_block(eb + 2, b)
                return 0

            lax.fori_loop(0, NBLK // 2, bb_body, 0)
            pltpu.sync_copy(acc_v, out_hbm.at[:, pl.ds(zb, ZH), :])

    return k(rf3, dtx3, drx3, apod3, kvec, acst)


def kernel(rf, t0, d_tx, d_rx, fs, c0, apod):
    n_angles, n_elements, n_samp = rf.shape
    kscale = (fs / c0).astype(jnp.float32)
    offs = (jnp.arange(n_angles, dtype=jnp.float32) * (EB * n_samp)
            - t0.astype(jnp.float32) * fs)
    acst = jnp.broadcast_to(offs[:, None], (n_angles, L))
    kvec = jnp.full((L,), kscale, jnp.float32)
    return _das_sc(rf, d_tx, d_rx, apod, kvec, acst)
